# baseline (speedup 1.0000x reference)
import jax
import jax.numpy as jnp
from jax import lax
from jax.experimental import pallas as pl
from jax.experimental.pallas import tpu as pltpu
from jax.experimental.pallas import tpu_sc as plsc

B = 16384
F = 26
V = 100000
D = 32

NC = 2
NS = 16
NW = NC * NS

V4 = V // 4
BPW = B // NW
CB = 128
NSL = 4


def _pack_body(t_ref, o_ref):
    t = t_ref[...]
    for q in range(4):
        o_ref[:, q * D:(q + 1) * D] = t[:, q * V4:(q + 1) * V4].T


def _pack(tbl_t, f0, nf):
    return pl.pallas_call(
        _pack_body,
        grid=(nf,),
        in_specs=[pl.BlockSpec((D, V), lambda f: (f0 + f, 0))],
        out_specs=pl.BlockSpec((V4, 128), lambda f: (f, 0)),
        out_shape=jax.ShapeDtypeStruct((nf * V4, 128), jnp.float32),
        compiler_params=pltpu.CompilerParams(
            vmem_limit_bytes=110 * 1024 * 1024),
    )(tbl_t)


def _gather_body(nf, f0, x_hbm, ptbl_hbm, out_hbm,
                 xk, idxb0, idxb1, idxb2, idxb3, laneb, gbuf, ebuf,
                 g0, g1, g2, g3, w0, w1, w2, w3):
    nt = nf * (BPW // CB)
    ngrp = nt // NSL
    c = lax.axis_index("c")
    s = lax.axis_index("s")
    wid = s * NC + c
    b0 = wid * BPW

    pltpu.sync_copy(x_hbm.at[pl.ds(b0 * F, BPW * F)], xk)

    iota = lax.iota(jnp.int32, 16)
    gsems = (g0, g1, g2, g3)
    wsems = (w0, w1, w2, w3)
    idxbs = (idxb0, idxb1, idxb2, idxb3)

    def build(t, slot):
        fl = t // NSL
        h = lax.rem(t, NSL)

        def grp(g, carry):
            j = h * CB + g * 16 + iota
            v = plsc.load_gather(xk, [f0 + fl + F * j])
            idxbs[slot][pl.ds(g * 16, 16)] = fl * V4 + lax.rem(v, V4)
            laneb[slot, pl.ds(g * 16, 16)] = lax.div(v, V4) * D
            return carry

        lax.fori_loop(0, CB // 16, grp, 0)

    def gstart(slot):
        pltpu.async_copy(ptbl_hbm.at[idxbs[slot]], gbuf.at[slot],
                         gsems[slot])

    def gwait(slot):
        pltpu.make_async_copy(ptbl_hbm.at[idxbs[slot]], gbuf.at[slot],
                              gsems[slot]).wait()

    def extract(slot):
        def grp(g, carry):
            j = g * 16 + iota
            lj = laneb[slot, pl.ds(g * 16, 16)]
            for d in range(D):
                ebuf[slot, d, pl.ds(g * 16, 16)] = \
                    plsc.load_gather(gbuf.at[slot], [j, lj + d])
            return carry

        lax.fori_loop(0, CB // 16, grp, 0)

    def wstart(t, slot):
        fl = t // NSL
        h = lax.rem(t, NSL)
        pltpu.async_copy(ebuf.at[slot],
                         out_hbm.at[fl, :, pl.ds(b0 + h * CB, CB)],
                         wsems[slot])

    def wwait(slot):
        pltpu.make_async_copy(ebuf.at[slot],
                              out_hbm.at[0, :, pl.ds(b0, CB)],
                              wsems[slot]).wait()

    for sl in range(NSL):
        build(sl, sl)
        gstart(sl)

    def group(gi, carry):
        for sl in range(NSL):
            t = NSL * gi + sl
            gwait(sl)

            @pl.when(gi >= 1)
            def _():
                wwait(sl)

            extract(sl)
            wstart(t, sl)

            @pl.when(gi < ngrp - 1)
            def _():
                build(t + NSL, sl)
                gstart(sl)

        return carry

    lax.fori_loop(0, ngrp, group, 0)
    for sl in range(NSL):
        wwait(sl)


def _gather(xf, ptbl, f0, nf):
    mesh = plsc.VectorSubcoreMesh(core_axis_name="c", subcore_axis_name="s",
                                  num_cores=NC, num_subcores=NS)

    def body(*refs):
        _gather_body(nf, f0, *refs)

    return pl.kernel(
        body,
        out_type=jax.ShapeDtypeStruct((nf, D, B), jnp.float32),
        mesh=mesh,
        scratch_types=[
            pltpu.VMEM((BPW * F,), jnp.int32),
            pltpu.VMEM((CB,), jnp.int32),
            pltpu.VMEM((CB,), jnp.int32),
            pltpu.VMEM((CB,), jnp.int32),
            pltpu.VMEM((CB,), jnp.int32),
            pltpu.VMEM((NSL, CB), jnp.int32),
            pltpu.VMEM((NSL, CB, 128), jnp.float32),
            pltpu.VMEM((NSL, D, CB), jnp.float32),
        ] + [pltpu.SemaphoreType.DMA] * 8,
        compiler_params=pltpu.CompilerParams(use_tc_tiling_on_sc=True,
                                             needs_layout_passes=False),
    )(xf, ptbl)


@jax.jit
def kernel(x, tables):
    xf = x.astype(jnp.int32).reshape(B * F)
    tbl_t = jnp.transpose(tables, (0, 2, 1)).reshape(F * D, V)
    halves = []
    for f0, nf in ((0, 8), (8, 8), (16, 8), (24, 2)):
        ptbl = _pack(tbl_t, f0, nf)
        halves.append(_gather(xf, ptbl, f0, nf))
    out_fdb = jnp.concatenate(halves, axis=0)
    return jnp.transpose(out_fdb, (2, 0, 1))
